# baseline (speedup 1.0000x reference)
import jax
import jax.numpy as jnp
from jax.experimental import pallas as pl

N = 10000; E = 320000; F = 128; H = 8; C = 16; HC = H * C; L = 3
N_FAM = 100; N_TYPE = 20; TE = 16; G = 64


def _gatv2(x, Wl, Wr, att, b, src, dst):
    n = x.shape[0]
    xl = (x @ Wl).reshape(n, H, C)
    xr = (x @ Wr).reshape(n, H, C)
    s = xl[src] + xr[dst]
    e = jax.nn.leaky_relu(s, 0.2)
    logits = jnp.sum(e * att[None], axis=-1)
    m = jax.ops.segment_max(logits, dst, num_segments=n)
    m = jnp.where(jnp.isfinite(m), m, 0.0)
    ex = jnp.exp(logits - m[dst])
    denom = jax.ops.segment_sum(ex, dst, num_segments=n)
    alpha = ex / (denom[dst] + 1e-16)
    out = jax.ops.segment_sum(xl[src] * alpha[..., None], dst, num_segments=n)
    return out.reshape(n, HC) + b


def _layernorm(z, g, be):
    mu = jnp.mean(z, axis=-1, keepdims=True)
    var = jnp.var(z, axis=-1, keepdims=True)
    return (z - mu) / jnp.sqrt(var + 1e-5) * g + be


def _readout_body(gcat_ref, Wf_ref, bf_ref, Wt_ref, bt_ref, o1_ref, o2_ref):
    gcat = gcat_ref[...]
    o1_ref[...] = gcat @ Wf_ref[...] + bf_ref[...][None, :]
    o2_ref[...] = gcat @ Wt_ref[...] + bt_ref[...][None, :]


def kernel(x, edge, batch, y_type, Wl0, Wr0, att0, b0, g0, be0, Wl1, Wr1, att1, b1, g1, be1, Wl2, Wr2, att2, b2, g2, be2, te_table, Wf, bf, Wt, bt):
    n = x.shape[0]
    loop = jnp.arange(n, dtype=edge.dtype)
    src = jnp.concatenate([edge[0], loop])
    dst = jnp.concatenate([edge[1], loop])
    params = [(Wl0, Wr0, att0, b0, g0, be0), (Wl1, Wr1, att1, b1, g1, be1), (Wl2, Wr2, att2, b2, g2, be2)]
    h = x
    for (Wl, Wr, att, b, g, be) in params:
        z = jax.nn.relu(_gatv2(h, Wl, Wr, att, b, src, dst))
        z = _layernorm(z, g, be)
        h = z + h
    sums = jax.ops.segment_sum(h, batch, num_segments=G)
    cnt = jax.ops.segment_sum(jnp.ones((n, 1), jnp.float32), batch, num_segments=G)
    gmean = sums / jnp.maximum(cnt, 1.0)
    gcat = jnp.concatenate([gmean, te_table[y_type]], axis=1)
    o1, o2 = pl.pallas_call(
        _readout_body,
        out_shape=(jax.ShapeDtypeStruct((G, N_FAM), jnp.float32),
                   jax.ShapeDtypeStruct((G, N_TYPE), jnp.float32)),
    )(gcat, Wf, bf, Wt, bt)
    return (o1, o2)


# speedup vs baseline: 26.6044x; 26.6044x over previous
import jax
import jax.numpy as jnp
from jax import lax
from jax.experimental import pallas as pl
from jax.experimental.pallas import tpu as pltpu
from jax.experimental.pallas import tpu_sc as plsc

N = 10000; E = 320000; F = 128; H = 8; C = 16; HC = H * C; L = 3
N_FAM = 100; N_TYPE = 20; TE = 16; G = 64

NPAD = 10240
NSC = 2
NSUB = 16
NTILES = NSC * NSUB
CHUNK = 64
EPT_CHUNKS = 162
EPT = EPT_CHUNKS * CHUNK
EPAD = NTILES * EPT
ROWS_PER_TILE = NPAD // NSUB
DW = 16


def _lin_body(h_ref, wl_ref, wr_ref, xl_ref, xr_ref):
    hb = h_ref[...]
    xl_ref[...] = jnp.dot(hb, wl_ref[...], preferred_element_type=jnp.float32)
    xr_ref[...] = jnp.dot(hb, wr_ref[...], preferred_element_type=jnp.float32)


def _lin(h, Wl, Wr):
    BLK = 512
    return pl.pallas_call(
        _lin_body,
        grid=(NPAD // BLK,),
        in_specs=[pl.BlockSpec((BLK, HC), lambda i: (i, 0)),
                  pl.BlockSpec((HC, HC), lambda i: (0, 0)),
                  pl.BlockSpec((HC, HC), lambda i: (0, 0))],
        out_specs=[pl.BlockSpec((BLK, HC), lambda i: (i, 0)),
                   pl.BlockSpec((BLK, HC), lambda i: (i, 0))],
        out_shape=(jax.ShapeDtypeStruct((NPAD, HC), jnp.float32),
                   jax.ShapeDtypeStruct((NPAD, HC), jnp.float32)),
    )(h, Wl, Wr)


def _edge_body(xl_hbm, xr_hbm, src_hbm, dst_hbm, att_hbm,
               s_out, ex_out,
               S_sh, sidx, didx, xlbuf, xrbuf, msgbuf, exT, attv,
               sem0, sem1):
    cid = lax.axis_index("c")
    sid = lax.axis_index("s")
    tid = cid * NSUB + sid
    row0 = sid * ROWS_PER_TILE

    pltpu.sync_copy(att_hbm, attv)

    lane = lax.iota(jnp.int32, 16)
    attvecs = [attv[pl.ds(h * C, C)] for h in range(H)]

    def _zmsg(e, carry):
        for q in range(HC // 16):
            msgbuf[e, pl.ds(q * 16, 16)] = jnp.zeros((16,), jnp.float32)
        exT[e, :] = jnp.zeros((16,), jnp.float32)
        return carry
    lax.fori_loop(0, CHUNK, _zmsg, 0)

    for j in range(ROWS_PER_TILE // CHUNK):
        pltpu.sync_copy(msgbuf, S_sh.at[pl.ds(row0 + j * CHUNK, CHUNK)])
    plsc.subcore_barrier()

    @pl.loop(0, EPT_CHUNKS)
    def _chunk(k):
        base = tid * EPT + k * CHUNK
        pltpu.async_copy(src_hbm.at[pl.ds(base, CHUNK)], sidx, sem0).wait()
        pltpu.async_copy(dst_hbm.at[pl.ds(base, CHUNK)], didx, sem1).wait()
        cp0 = pltpu.async_copy(xl_hbm.at[sidx], xlbuf, sem0)
        cp1 = pltpu.async_copy(xr_hbm.at[didx], xrbuf, sem1)
        cp0.wait()
        cp1.wait()

        def _edge1(e, carry2):
            for h in range(H):
                xlv = xlbuf[e, pl.ds(h * C, C)]
                xrv = xrbuf[e, pl.ds(h * C, C)]
                s = xlv + xrv
                lrel = jnp.where(s > 0, s, 0.2 * s)
                msgbuf[e, pl.ds(h * C, C)] = lrel * attvecs[h]
            return carry2
        lax.fori_loop(0, CHUNK, _edge1, 0)

        def _group(g, carry2):
            rows = g * 16 + lane
            for h in range(H):
                acc = jnp.zeros((16,), jnp.float32)
                for c in range(C):
                    dfull = jnp.full((16,), h * C + c, jnp.int32)
                    acc = acc + plsc.load_gather(msgbuf, [rows, dfull])
                plsc.store_scatter(exT, [rows, jnp.full((16,), h, jnp.int32)],
                                   jnp.exp(acc))
            return carry2
        lax.fori_loop(0, CHUNK // 16, _group, 0)

        def _edge3(e, carry2):
            exv = exT[e, :]
            for h in range(H):
                msgbuf[e, pl.ds(h * C, C)] = xlbuf[e, pl.ds(h * C, C)] * exv[h]
            return carry2
        lax.fori_loop(0, CHUNK, _edge3, 0)

        pltpu.sync_copy(msgbuf, S_sh.at[didx], add=True)
        pltpu.sync_copy(exT, ex_out.at[pl.ds(base, CHUNK)])

    plsc.subcore_barrier()

    for j in range(ROWS_PER_TILE // CHUNK):
        r = row0 + j * CHUNK
        o = cid * NPAD + r
        pltpu.sync_copy(S_sh.at[pl.ds(r, CHUNK)], msgbuf)
        pltpu.sync_copy(msgbuf, s_out.at[pl.ds(o, CHUNK)])


def _sc_edge(xl, xr, srcp, dstp, attf):
    mesh = plsc.VectorSubcoreMesh(core_axis_name="c", subcore_axis_name="s")
    f = pl.kernel(
        _edge_body,
        out_type=(jax.ShapeDtypeStruct((NSC * NPAD, HC), jnp.float32),
                  jax.ShapeDtypeStruct((EPAD, DW), jnp.float32)),
        mesh=mesh,
        compiler_params=pltpu.CompilerParams(needs_layout_passes=False),
        scratch_types=[
            pltpu.VMEM_SHARED((NPAD, HC), jnp.float32),
            pltpu.VMEM((CHUNK,), jnp.int32),
            pltpu.VMEM((CHUNK,), jnp.int32),
            pltpu.VMEM((CHUNK, HC), jnp.float32),
            pltpu.VMEM((CHUNK, HC), jnp.float32),
            pltpu.VMEM((CHUNK, HC), jnp.float32),
            pltpu.VMEM((CHUNK, DW), jnp.float32),
            pltpu.VMEM((HC,), jnp.float32),
            pltpu.SemaphoreType.DMA,
            pltpu.SemaphoreType.DMA,
        ],
    )
    return f(xl, xr, srcp, dstp, attf)


def _denom_body(ex_hbm, dst_hbm, d_out,
                D_sh, didx, exbuf, stg, sem0, sem1):
    cid = lax.axis_index("c")
    sid = lax.axis_index("s")
    tid = cid * NSUB + sid
    row0 = sid * ROWS_PER_TILE

    def _zstg(e, carry):
        for q in range(HC // 16):
            stg[e, pl.ds(q * 16, 16)] = jnp.zeros((16,), jnp.float32)
        return carry
    lax.fori_loop(0, CHUNK, _zstg, 0)

    for j in range(ROWS_PER_TILE // CHUNK):
        pltpu.sync_copy(stg, D_sh.at[pl.ds(row0 + j * CHUNK, CHUNK)])
    plsc.subcore_barrier()

    @pl.loop(0, EPT_CHUNKS)
    def _chunk(k):
        base = tid * EPT + k * CHUNK
        pltpu.async_copy(dst_hbm.at[pl.ds(base, CHUNK)], didx, sem0).wait()
        pltpu.async_copy(ex_hbm.at[pl.ds(base, CHUNK)], exbuf, sem1).wait()

        def _edge(e, carry2):
            stg[e, pl.ds(0, DW)] = exbuf[e, :]
            return carry2
        lax.fori_loop(0, CHUNK, _edge, 0)

        pltpu.sync_copy(stg, D_sh.at[didx], add=True)

    plsc.subcore_barrier()

    for j in range(ROWS_PER_TILE // CHUNK):
        r = row0 + j * CHUNK
        o = cid * NPAD + r
        pltpu.sync_copy(D_sh.at[pl.ds(r, CHUNK)], stg)
        pltpu.sync_copy(stg, d_out.at[pl.ds(o, CHUNK)])


def _sc_denom(exe, dstp):
    mesh = plsc.VectorSubcoreMesh(core_axis_name="c", subcore_axis_name="s")
    f = pl.kernel(
        _denom_body,
        out_type=jax.ShapeDtypeStruct((NSC * NPAD, HC), jnp.float32),
        mesh=mesh,
        compiler_params=pltpu.CompilerParams(needs_layout_passes=False),
        scratch_types=[
            pltpu.VMEM_SHARED((NPAD, HC), jnp.float32),
            pltpu.VMEM((CHUNK,), jnp.int32),
            pltpu.VMEM((CHUNK, DW), jnp.float32),
            pltpu.VMEM((CHUNK, HC), jnp.float32),
            pltpu.SemaphoreType.DMA,
            pltpu.SemaphoreType.DMA,
        ],
    )
    return f(exe, dstp)


def _post_body(s0_ref, s1_ref, d0_ref, d1_ref, b_ref, g_ref, be_ref, h_ref,
               o_ref):
    BLK = s0_ref.shape[0]
    S = s0_ref[...] + s1_ref[...]
    Dsum = d0_ref[...] + d1_ref[...]
    r = lax.broadcasted_iota(jnp.int32, (HC, HC), 0)
    d = lax.broadcasted_iota(jnp.int32, (HC, HC), 1)
    p2 = (r == d // C).astype(jnp.float32)
    dex = jnp.dot(Dsum, p2, preferred_element_type=jnp.float32)
    z = jnp.maximum(S / (dex + 1e-16) + b_ref[...], 0.0)
    mu = jnp.mean(z, axis=-1, keepdims=True)
    var = jnp.mean((z - mu) ** 2, axis=-1, keepdims=True)
    zn = (z - mu) / jnp.sqrt(var + 1e-5) * g_ref[...] + be_ref[...]
    hn = zn + h_ref[...]
    rows = pl.program_id(0) * BLK + lax.broadcasted_iota(jnp.int32, (BLK, HC), 0)
    o_ref[...] = jnp.where(rows < N, hn, 0.0)


def _post(S2, D2, b, g, be, h):
    BLK = 512
    nb = NPAD // BLK
    return pl.pallas_call(
        _post_body,
        grid=(nb,),
        in_specs=[pl.BlockSpec((BLK, HC), lambda i: (i, 0)),
                  pl.BlockSpec((BLK, HC), lambda i: (i + nb, 0)),
                  pl.BlockSpec((BLK, HC), lambda i: (i, 0)),
                  pl.BlockSpec((BLK, HC), lambda i: (i + nb, 0)),
                  pl.BlockSpec((1, HC), lambda i: (0, 0)),
                  pl.BlockSpec((1, HC), lambda i: (0, 0)),
                  pl.BlockSpec((1, HC), lambda i: (0, 0)),
                  pl.BlockSpec((BLK, HC), lambda i: (i, 0))],
        out_specs=pl.BlockSpec((BLK, HC), lambda i: (i, 0)),
        out_shape=jax.ShapeDtypeStruct((NPAD, HC), jnp.float32),
    )(S2, S2, D2, D2, b, g, be, h)


def _pool_body(h_ref, batch_ref, y_ref, te_ref, wf1_ref, wf2_ref, bf_ref,
               wt1_ref, wt2_ref, bt_ref, o1_ref, o2_ref):
    hb = h_ref[...]
    bb = batch_ref[...]
    gids = lax.broadcasted_iota(jnp.int32, (G, N), 0)
    oh = (bb == gids).astype(jnp.float32)
    sums = jnp.dot(oh, hb, preferred_element_type=jnp.float32)
    cnt = jnp.sum(oh, axis=1, keepdims=True)
    gmean = sums / jnp.maximum(cnt, 1.0)
    yv = y_ref[...]
    tids = lax.broadcasted_iota(jnp.int32, (G, N_TYPE), 1)
    ohy = (yv == tids).astype(jnp.float32)
    emb = jnp.dot(ohy, te_ref[...], preferred_element_type=jnp.float32)
    o1_ref[...] = (jnp.dot(gmean, wf1_ref[...], preferred_element_type=jnp.float32)
                   + jnp.dot(emb, wf2_ref[...], preferred_element_type=jnp.float32)
                   + bf_ref[...])
    o2_ref[...] = (jnp.dot(gmean, wt1_ref[...], preferred_element_type=jnp.float32)
                   + jnp.dot(emb, wt2_ref[...], preferred_element_type=jnp.float32)
                   + bt_ref[...])


def _pool(h, batch2d, y2d, te_table, Wf1, Wf2, bf, Wt1, Wt2, bt):
    return pl.pallas_call(
        _pool_body,
        out_shape=(jax.ShapeDtypeStruct((G, N_FAM), jnp.float32),
                   jax.ShapeDtypeStruct((G, N_TYPE), jnp.float32)),
    )(h, batch2d, y2d, te_table, Wf1, Wf2, bf, Wt1, Wt2, bt)


def kernel(x, edge, batch, y_type, Wl0, Wr0, att0, b0, g0, be0, Wl1, Wr1, att1,
           b1, g1, be1, Wl2, Wr2, att2, b2, g2, be2, te_table, Wf, bf, Wt, bt):
    h = jnp.pad(x, ((0, NPAD - N), (0, 0)))
    loop = jnp.arange(N, dtype=edge.dtype)
    pad_idx = jnp.full((EPAD - E - N,), N, edge.dtype)
    srcp = jnp.concatenate([edge[0], loop, pad_idx])
    dstp = jnp.concatenate([edge[1], loop, pad_idx])

    params = [(Wl0, Wr0, att0, b0, g0, be0),
              (Wl1, Wr1, att1, b1, g1, be1),
              (Wl2, Wr2, att2, b2, g2, be2)]
    for (Wl, Wr, att, b, g, be) in params:
        xl, xr = _lin(h, Wl, Wr)
        S2, exe = _sc_edge(xl, xr, srcp, dstp, att.reshape(HC))
        D2 = _sc_denom(exe, dstp)
        h = _post(S2, D2, b.reshape(1, HC), g.reshape(1, HC),
                  be.reshape(1, HC), h)

    o1, o2 = _pool(h[:N], batch.reshape(1, N), y_type.reshape(G, 1), te_table,
                   Wf[:HC], Wf[HC:], bf.reshape(1, N_FAM),
                   Wt[:HC], Wt[HC:], bt.reshape(1, N_TYPE))
    return (o1, o2)
